# baseline (speedup 1.0000x reference)
import math

import jax
import jax.numpy as jnp
from jax import lax
from jax.experimental import pallas as pl
from jax.experimental.pallas import tpu as pltpu

D = 256
J = 32
T = 128
DEPTH = 6
H = 8
DH = D // H
ROWS = T * J
CHUNK = 512
SCALE = 1.0 / math.sqrt(DH)


def _gelu(x):
    return 0.5 * x * (1.0 + lax.erf(x * (1.0 / math.sqrt(2.0))))


def _dot(a, b):
    return lax.dot_general(a, b, (((1,), (0,)), ((), ())),
                           preferred_element_type=jnp.float32)


def _mm(a, b):
    return _dot(a.astype(jnp.bfloat16), b.astype(jnp.bfloat16))


def _body(z_ref, t_ref, h_ref, tW1, tb1, tW2, tb2, cW1, cb1, cW2, cb2,
          Wqkv, bqkv, Wo, bo, W1, b1, W2, b2,
          tlng, tlnb, convW, convb, oW, ob, out_ref, x_ref):
    tval = t_ref[0, 0, 0]
    half = D // 2
    i_half = lax.broadcasted_iota(jnp.int32, (1, half), 1).astype(jnp.float32)
    freqs = jnp.exp(i_half * (-math.log(10000.0) / half))
    args = tval * freqs
    temb = jnp.concatenate([jnp.cos(args), jnp.sin(args)], axis=-1)
    temb = _mm(_gelu(_mm(temb, tW1[:]) + tb1[:]), tW2[:]) + tb2[:]

    hrow = h_ref[0]
    film = _mm(_gelu(_mm(hrow, cW1[:]) + cb1[:]), cW2[:]) + cb2[:]
    gamma = film[:, :D]
    beta = film[:, D:]

    x_ref[:] = (z_ref[0] + temb) * (1.0 + gamma) + beta

    rblk = lax.broadcasted_iota(jnp.int32, (D, D), 0) // DH
    cblk = lax.broadcasted_iota(jnp.int32, (D, D), 1) // DH
    bd = (rblk == cblk).astype(jnp.bfloat16)
    jmod = lax.broadcasted_iota(jnp.int32, (CHUNK, D), 0) % J
    left_ok = (jmod != 0).astype(jnp.bfloat16)
    right_ok = (jmod != (J - 1)).astype(jnp.bfloat16)

    def norm(x):
        m = jnp.mean(x, axis=-1, keepdims=True)
        xc = x - m
        v = jnp.mean(xc * xc, axis=-1, keepdims=True)
        return xc * lax.rsqrt(v + 1e-5)

    for i in range(DEPTH):
        for c in range(ROWS // CHUNK):
            sl = pl.ds(c * CHUNK, CHUNK)
            xc = x_ref[sl, :]
            qkv = _mm(norm(xc), Wqkv[i]) + bqkv[i:i + 1]
            q = ((qkv[:, :D] * SCALE)).astype(jnp.bfloat16)
            k = qkv[:, D:2 * D].astype(jnp.bfloat16)
            v = qkv[:, 2 * D:].astype(jnp.bfloat16)
            zrow = jnp.zeros((1, D), jnp.bfloat16)
            kp = jnp.concatenate([zrow, k[:-1]], axis=0)
            kn = jnp.concatenate([k[1:], zrow], axis=0)
            es = jnp.exp(_dot(q * k, bd).astype(jnp.bfloat16))
            el = jnp.exp(_dot(q * kp, bd).astype(jnp.bfloat16)) * left_ok
            er = jnp.exp(_dot(q * kn, bd).astype(jnp.bfloat16)) * right_ok
            vp = jnp.concatenate([zrow, v[:-1]], axis=0)
            vn = jnp.concatenate([v[1:], zrow], axis=0)
            o = (es * v + el * vp + er * vn) / (es + el + er)
            x_ref[sl, :] = xc + _dot(o, Wo[i]) + bo[i:i + 1]

        for c in range(ROWS // CHUNK):
            sl = pl.ds(c * CHUNK, CHUNK)
            xc = x_ref[sl, :]
            hid = _gelu((_mm(norm(xc), W1[i]) + b1[i:i + 1])
                        .astype(jnp.bfloat16))
            x_ref[sl, :] = xc + _dot(hid, W2[i]) + b2[i:i + 1]

        y = (norm(x_ref[:]) * tlng[i:i + 1] + tlnb[i:i + 1]) \
            .astype(jnp.bfloat16)
        a0 = _dot(y, convW[i, 0])
        a1 = _dot(y, convW[i, 1])
        a2 = _dot(y, convW[i, 2])
        zfrm = jnp.zeros((J, D), jnp.float32)
        prev = jnp.concatenate([zfrm, a0[:-J]], axis=0)
        nxt = jnp.concatenate([a2[J:], zfrm], axis=0)
        x_ref[:] = x_ref[:] + _gelu(
            (prev + a1 + nxt + convb[i:i + 1]).astype(jnp.bfloat16))

    out_ref[0] = _mm(x_ref[:], oW[:]) + ob[:]


def kernel(z_t, t, h, time_W1, time_b1, time_W2, time_b2, cond_W1, cond_b1,
           cond_W2, cond_b2, g_ln1_g, g_ln1_b, g_Wqkv, g_bqkv, g_Wo, g_bo,
           g_ln2_g, g_ln2_b, g_W1, g_b1, g_W2, g_b2, t_ln_g, t_ln_b,
           t_conv_W, t_conv_b, out_W, out_b):
    B = z_t.shape[0]
    bf = jnp.bfloat16
    z2 = z_t.reshape(B, ROWS, D)
    t3 = t.astype(jnp.float32).reshape(B, 1, 1)
    h3 = h.reshape(B, 1, D)

    Wqkv_eff = (g_ln1_g[:, :, None] * g_Wqkv).astype(bf)
    bqkv_eff = jnp.einsum('ld,ldk->lk', g_ln1_b, g_Wqkv) + g_bqkv
    W1_eff = (g_ln2_g[:, :, None] * g_W1).astype(bf)
    b1_eff = jnp.einsum('ld,ldk->lk', g_ln2_b, g_W1) + g_b1

    def row(x):
        return x.reshape(1, -1)

    full = lambda a: pl.BlockSpec(a.shape, lambda b: (0,) * a.ndim)
    args = [
        z2, t3, h3,
        time_W1, row(time_b1), time_W2, row(time_b2),
        cond_W1, row(cond_b1), cond_W2, row(cond_b2),
        Wqkv_eff, bqkv_eff, g_Wo.astype(bf), g_bo,
        W1_eff, b1_eff, g_W2.astype(bf), g_b2,
        t_ln_g, t_ln_b, t_conv_W.astype(bf), t_conv_b,
        out_W.astype(bf), row(out_b),
    ]
    def _invoke(*a):
        nb = a[0].shape[0]
        in_specs = [
            pl.BlockSpec((1, ROWS, D), lambda b: (b, 0, 0)),
            pl.BlockSpec((1, 1, 1), lambda b: (b, 0, 0)),
            pl.BlockSpec((1, 1, D), lambda b: (b, 0, 0)),
        ] + [full(x) for x in a[3:]]
        return pl.pallas_call(
            _body,
            grid=(nb,),
            in_specs=in_specs,
            out_specs=pl.BlockSpec((1, ROWS, D), lambda b: (b, 0, 0)),
            out_shape=jax.ShapeDtypeStruct((nb, ROWS, D), jnp.float32),
            scratch_shapes=[pltpu.VMEM((ROWS, D), jnp.float32)],
            compiler_params=pltpu.CompilerParams(
                dimension_semantics=("arbitrary",),
                vmem_limit_bytes=100 * 1024 * 1024,
            ),
        )(*a)

    out = _invoke(*args)
    return out.reshape(B, T, J, D)


# speedup vs baseline: 1.0220x; 1.0220x over previous
import math

import jax
import jax.numpy as jnp
from jax import lax
from jax.experimental import pallas as pl
from jax.experimental.pallas import tpu as pltpu

D = 256
J = 32
T = 128
DEPTH = 6
H = 8
DH = D // H
ROWS = T * J
CHUNK = 1024
SCALE = 1.0 / math.sqrt(DH)


def _gelu(x):
    return 0.5 * x * (1.0 + lax.erf(x * (1.0 / math.sqrt(2.0))))


def _dot(a, b, out_dtype=jnp.float32):
    return lax.dot_general(a, b, (((1,), (0,)), ((), ())),
                           preferred_element_type=out_dtype)


def _mm(a, b):
    return _dot(a.astype(jnp.bfloat16), b.astype(jnp.bfloat16))


def _body(z_ref, t_ref, h_ref, tW1, tb1, tW2, tb2, cW1, cb1, cW2, cb2,
          Wqkv, bqkv, Wo, bo, W1, b1, W2, b2,
          tlng, tlnb, convW, convb, oW, ob, out_ref, x_ref):
    tval = t_ref[0, 0, 0]
    half = D // 2
    i_half = lax.broadcasted_iota(jnp.int32, (1, half), 1).astype(jnp.float32)
    freqs = jnp.exp(i_half * (-math.log(10000.0) / half))
    args = tval * freqs
    temb = jnp.concatenate([jnp.cos(args), jnp.sin(args)], axis=-1)
    temb = _mm(_gelu(_mm(temb, tW1[:]) + tb1[:]), tW2[:]) + tb2[:]

    hrow = h_ref[0]
    film = _mm(_gelu(_mm(hrow, cW1[:]) + cb1[:]), cW2[:]) + cb2[:]
    gamma = film[:, :D]
    beta = film[:, D:]

    x_ref[:] = (z_ref[0] + temb) * (1.0 + gamma) + beta

    rblk = lax.broadcasted_iota(jnp.int32, (D, D), 0) // DH
    cblk = lax.broadcasted_iota(jnp.int32, (D, D), 1) // DH
    bd = (rblk == cblk).astype(jnp.bfloat16)
    jmod = lax.broadcasted_iota(jnp.int32, (CHUNK, D), 0) % J
    left_ok = (jmod != 0).astype(jnp.bfloat16)
    right_ok = (jmod != (J - 1)).astype(jnp.bfloat16)

    def norm(x):
        m = jnp.mean(x, axis=-1, keepdims=True)
        xc = x - m
        v = jnp.mean(xc * xc, axis=-1, keepdims=True)
        return xc * lax.rsqrt(v + 1e-5)

    for i in range(DEPTH):
        for c in range(ROWS // CHUNK):
            sl = pl.ds(c * CHUNK, CHUNK)
            xc = x_ref[sl, :]
            qkv = _dot(norm(xc).astype(jnp.bfloat16), Wqkv[i]) \
                + bqkv[i:i + 1]
            q = qkv[:, :D].astype(jnp.bfloat16)
            k = qkv[:, D:2 * D].astype(jnp.bfloat16)
            v = qkv[:, 2 * D:].astype(jnp.bfloat16)
            zrow = jnp.zeros((1, D), jnp.bfloat16)
            kp = jnp.concatenate([zrow, k[:-1]], axis=0)
            kn = jnp.concatenate([k[1:], zrow], axis=0)
            es = jnp.exp(_dot(q * k, bd).astype(jnp.bfloat16))
            el = jnp.exp(_dot(q * kp, bd).astype(jnp.bfloat16)) * left_ok
            er = jnp.exp(_dot(q * kn, bd).astype(jnp.bfloat16)) * right_ok
            vp = jnp.concatenate([zrow, v[:-1]], axis=0)
            vn = jnp.concatenate([v[1:], zrow], axis=0)
            o = (es * v + el * vp + er * vn) / (es + el + er)
            x_ref[sl, :] = xc + _dot(o, Wo[i]) + bo[i:i + 1]

        for c in range(ROWS // CHUNK):
            sl = pl.ds(c * CHUNK, CHUNK)
            xc = x_ref[sl, :]
            hid = _gelu((_dot(norm(xc).astype(jnp.bfloat16), W1[i])
                         + b1[i:i + 1]).astype(jnp.bfloat16))
            x_ref[sl, :] = xc + _dot(hid, W2[i]) + b2[i:i + 1]

        y = (norm(x_ref[:]) * tlng[i:i + 1] + tlnb[i:i + 1]) \
            .astype(jnp.bfloat16)
        a0 = _dot(y, convW[i, 0])
        a1 = _dot(y, convW[i, 1])
        a2 = _dot(y, convW[i, 2])
        zfrm = jnp.zeros((J, D), jnp.float32)
        prev = jnp.concatenate([zfrm, a0[:-J]], axis=0)
        nxt = jnp.concatenate([a2[J:], zfrm], axis=0)
        x_ref[:] = x_ref[:] + _gelu(
            (prev + a1 + nxt + convb[i:i + 1]).astype(jnp.bfloat16))

    out_ref[0] = _mm(x_ref[:], oW[:]) + ob[:]


def kernel(z_t, t, h, time_W1, time_b1, time_W2, time_b2, cond_W1, cond_b1,
           cond_W2, cond_b2, g_ln1_g, g_ln1_b, g_Wqkv, g_bqkv, g_Wo, g_bo,
           g_ln2_g, g_ln2_b, g_W1, g_b1, g_W2, g_b2, t_ln_g, t_ln_b,
           t_conv_W, t_conv_b, out_W, out_b):
    B = z_t.shape[0]
    bf = jnp.bfloat16
    z2 = z_t.reshape(B, ROWS, D)
    t3 = t.astype(jnp.float32).reshape(B, 1, 1)
    h3 = h.reshape(B, 1, D)

    qscale = jnp.concatenate(
        [jnp.full((D,), SCALE, jnp.float32), jnp.ones((2 * D,), jnp.float32)])
    Wqkv_eff = (g_ln1_g[:, :, None] * g_Wqkv * qscale).astype(bf)
    bqkv_eff = ((jnp.einsum('ld,ldk->lk', g_ln1_b, g_Wqkv) + g_bqkv)
                * qscale).astype(bf)
    W1_eff = (g_ln2_g[:, :, None] * g_W1).astype(bf)
    b1_eff = (jnp.einsum('ld,ldk->lk', g_ln2_b, g_W1) + g_b1).astype(bf)

    def row(x):
        return x.reshape(1, -1)

    full = lambda a: pl.BlockSpec(a.shape, lambda b: (0,) * a.ndim)
    args = [
        z2, t3, h3,
        time_W1, row(time_b1), time_W2, row(time_b2),
        cond_W1, row(cond_b1), cond_W2, row(cond_b2),
        Wqkv_eff, bqkv_eff, g_Wo.astype(bf), g_bo,
        W1_eff, b1_eff, g_W2.astype(bf), g_b2,
        t_ln_g, t_ln_b, t_conv_W.astype(bf), t_conv_b.astype(bf),
        out_W.astype(bf), row(out_b),
    ]
    def _invoke(*a):
        nb = a[0].shape[0]
        in_specs = [
            pl.BlockSpec((1, ROWS, D), lambda b: (b, 0, 0)),
            pl.BlockSpec((1, 1, 1), lambda b: (b, 0, 0)),
            pl.BlockSpec((1, 1, D), lambda b: (b, 0, 0)),
        ] + [full(x) for x in a[3:]]
        return pl.pallas_call(
            _body,
            grid=(nb,),
            in_specs=in_specs,
            out_specs=pl.BlockSpec((1, ROWS, D), lambda b: (b, 0, 0)),
            out_shape=jax.ShapeDtypeStruct((nb, ROWS, D), jnp.float32),
            scratch_shapes=[pltpu.VMEM((ROWS, D), jnp.float32)],
            compiler_params=pltpu.CompilerParams(
                dimension_semantics=("arbitrary",),
                vmem_limit_bytes=100 * 1024 * 1024,
            ),
        )(*a)

    out = _invoke(*args)
    return out.reshape(B, T, J, D)
